# baseline (speedup 1.0000x reference)
import jax
import jax.numpy as jnp
from jax import lax
from jax.experimental import pallas as pl

_ROWS = 3584
_LG = 128
_NG = 6


def _rev_body(x_ref, p_ref, o_ref):
    p = p_ref[...]
    for g in range(_NG):
        o_ref[:, (_NG - 1 - g) * _LG:(_NG - g) * _LG] = lax.dot(
            x_ref[:, g * _LG:(g + 1) * _LG],
            p,
            precision=lax.Precision.HIGHEST,
            preferred_element_type=jnp.float32,
        )


def kernel(input):
    B, C, H, W = input.shape
    x2 = input.transpose(0, 2, 3, 1).reshape(B * H * W, C)
    nblk = (B * H * W) // _ROWS
    p = jnp.flip(jnp.eye(_LG, dtype=jnp.float32), axis=1)
    out2 = pl.pallas_call(
        _rev_body,
        grid=(nblk,),
        in_specs=[
            pl.BlockSpec((_ROWS, C), lambda i: (i, 0)),
            pl.BlockSpec((_LG, _LG), lambda i: (0, 0)),
        ],
        out_specs=pl.BlockSpec((_ROWS, C), lambda i: (i, 0)),
        out_shape=jax.ShapeDtypeStruct((B * H * W, C), jnp.float32),
    )(x2, p)
    return out2.reshape(B, H, W, C).transpose(0, 3, 1, 2)
